# baseline (speedup 1.0000x reference)
import functools

import jax
import jax.numpy as jnp
from jax import lax
from jax.experimental import pallas as pl
from jax.experimental.pallas import tpu as pltpu
from jax.experimental.pallas import tpu_sc as plsc

LANES = 16
NC = 2
NS = 16
NW = NC * NS


@functools.lru_cache(maxsize=None)
def _build(n, v_rows, d):
    pad = v_rows - 1
    start = v_rows
    per_w = n // NW
    c_rows = 1280
    s_rows = 896
    d_rows = c_rows - s_rows
    n_chunks = per_w // c_rows
    assert n_chunks % 2 == 0 and per_w % c_rows == 0
    mesh = plsc.VectorSubcoreMesh(core_axis_name="c", subcore_axis_name="s")

    @functools.partial(
        pl.kernel,
        mesh=mesh,
        out_type=jax.ShapeDtypeStruct((n, d), jnp.float32),
        compiler_params=pltpu.CompilerParams(use_tc_tiling_on_sc=False),
        scratch_types=[
            pltpu.VMEM((2, c_rows), jnp.int32),
            pltpu.VMEM((2, c_rows), jnp.int32),
            pltpu.VMEM((2, s_rows, d), jnp.float32),
            pltpu.VMEM_SHARED((NW, 2, d_rows, d), jnp.float32),
            pltpu.VMEM((2, d), jnp.float32),
            pltpu.SemaphoreType.DMA,
            pltpu.SemaphoreType.DMA,
            pltpu.SemaphoreType.DMA,
            pltpu.SemaphoreType.DMA,
            pltpu.SemaphoreType.DMA,
            pltpu.SemaphoreType.DMA,
        ],
    )
    def body(idx_hbm, table_hbm, beg_hbm, out_hbm, idx_raw, idx_safe, rows,
             shrows, beg_v, gsem0, gsem1, dsem0, dsem1, wsem0, wsem1):
        gsem = (gsem0, gsem1)
        dsem = (dsem0, dsem1)
        wsem = (wsem0, wsem1)
        wid = lax.axis_index("s") * NC + lax.axis_index("c")
        base = wid * per_w
        pltpu.sync_copy(beg_hbm, beg_v)

        def prep(gl, b):
            gc = jnp.minimum(gl, n_chunks - 1)
            off = base + gc * c_rows
            pltpu.sync_copy(idx_hbm.at[pl.ds(off, c_rows)], idx_raw.at[b])

            def remap(i, macc):
                v = idx_raw[b, pl.ds(i * LANES, LANES)]
                idx_safe[b, pl.ds(i * LANES, LANES)] = jnp.minimum(v, pad)
                return jnp.maximum(macc, v)

            macc = lax.fori_loop(0, c_rows // LANES, remap,
                                 jnp.zeros((LANES,), jnp.int32))
            pltpu.async_copy(table_hbm.at[idx_safe.at[b, pl.ds(0, s_rows)]],
                             rows.at[b], gsem[b])

            def fetch(j, c2):
                sv = idx_safe[b, pl.ds(s_rows + j * LANES, LANES)]
                for lane in range(LANES):
                    r = j * LANES + lane
                    pltpu.async_copy(table_hbm.at[pl.ds(sv[lane], 1)],
                                     shrows.at[wid, b, pl.ds(r, 1)], dsem[b])
                return c2

            lax.fori_loop(0, d_rows // LANES, fetch, 0)
            mxs = macc[0]
            for k in range(1, LANES):
                mxs = jnp.maximum(mxs, macc[k])
            return mxs

        def fixup(b, mxs):
            @pl.when(mxs >= start)
            def _fix():
                b0 = [beg_v[0, pl.ds(k * LANES, LANES)]
                      for k in range(d // LANES)]
                b1 = [beg_v[1, pl.ds(k * LANES, LANES)]
                      for k in range(d // LANES)]

                def fv(j, c2):
                    vraw = idx_raw[b, pl.ds(j * LANES, LANES)]
                    for lane in range(LANES):
                        sv = vraw[lane]
                        r = j * LANES + lane

                        @pl.when(sv == start)
                        def _s(r=r):
                            @pl.when(r < s_rows)
                            def _a():
                                for k in range(d // LANES):
                                    rows[b, r, pl.ds(k * LANES, LANES)] = \
                                        b0[k]

                            @pl.when(r >= s_rows)
                            def _b():
                                pltpu.sync_copy(
                                    beg_v.at[pl.ds(0, 1)],
                                    shrows.at[wid, b,
                                              pl.ds(r - s_rows, 1)])

                        @pl.when(sv == start + 1)
                        def _e(r=r):
                            @pl.when(r < s_rows)
                            def _a():
                                for k in range(d // LANES):
                                    rows[b, r, pl.ds(k * LANES, LANES)] = \
                                        b1[k]

                            @pl.when(r >= s_rows)
                            def _b():
                                pltpu.sync_copy(
                                    beg_v.at[pl.ds(1, 1)],
                                    shrows.at[wid, b,
                                              pl.ds(r - s_rows, 1)])
                    return c2

                lax.fori_loop(0, c_rows // LANES, fv, 0)

        mx0 = prep(0, 0)

        @pl.loop(0, n_chunks, step=2, init_carry=mx0)
        def _pipe(g, mx_cur):
            for b in range(2):
                gl = g + b
                nb = 1 - b

                @pl.when(gl > 0)
                def _drain(gl=gl, nb=nb):
                    off_prev = base + (gl - 1) * c_rows
                    pltpu.make_async_copy(
                        rows.at[nb],
                        out_hbm.at[pl.ds(off_prev, s_rows)],
                        wsem[nb]).wait()
                    pltpu.make_async_copy(
                        shrows.at[wid, nb],
                        out_hbm.at[pl.ds(off_prev + s_rows, d_rows)],
                        wsem[nb]).wait()

                mx_next = prep(gl + 1, nb)
                pltpu.make_async_copy(
                    table_hbm.at[idx_safe.at[b, pl.ds(0, s_rows)]],
                    rows.at[b], gsem[b]).wait()
                pltpu.make_async_copy(table_hbm.at[pl.ds(0, d_rows)],
                                      shrows.at[wid, b], dsem[b]).wait()
                fixup(b, mx_cur)
                off = base + gl * c_rows
                pltpu.async_copy(rows.at[b],
                                 out_hbm.at[pl.ds(off, s_rows)], wsem[b])
                pltpu.async_copy(shrows.at[wid, b],
                                 out_hbm.at[pl.ds(off + s_rows, d_rows)],
                                 wsem[b])
                mx_cur = mx_next
            return mx_cur

        pltpu.make_async_copy(table_hbm.at[idx_safe.at[0, pl.ds(0, s_rows)]],
                              rows.at[0], gsem[0]).wait()
        pltpu.make_async_copy(table_hbm.at[pl.ds(0, d_rows)],
                              shrows.at[wid, 0], dsem[0]).wait()
        off_last = base + (n_chunks - 1) * c_rows
        pltpu.make_async_copy(rows.at[1],
                              out_hbm.at[pl.ds(off_last, s_rows)],
                              wsem[1]).wait()
        pltpu.make_async_copy(shrows.at[wid, 1],
                              out_hbm.at[pl.ds(off_last + s_rows, d_rows)],
                              wsem[1]).wait()

    return body


def kernel(idxes, table, beg_end):
    b, h = idxes.shape
    v_rows, d = table.shape
    n = b * h
    flat = idxes.reshape(n)
    out = _build(n, v_rows, d)(flat, table, beg_end)
    return out.reshape(b, h, d)


# speedup vs baseline: 1.4213x; 1.4213x over previous
import functools

import jax
import jax.numpy as jnp
from jax import lax
from jax.experimental import pallas as pl
from jax.experimental.pallas import tpu as pltpu
from jax.experimental.pallas import tpu_sc as plsc

LANES = 16
NC = 2
NS = 16
NW = NC * NS


@functools.lru_cache(maxsize=None)
def _build(n, v_rows, d):
    pad = v_rows - 1
    start = v_rows
    per_w = n // NW
    c_rows = 1280
    KSTR = 4
    n_chunks = per_w // c_rows
    assert n_chunks % 2 == 0 and per_w % c_rows == 0
    mesh = plsc.VectorSubcoreMesh(core_axis_name="c", subcore_axis_name="s")

    @functools.partial(
        pl.kernel,
        mesh=mesh,
        out_type=jax.ShapeDtypeStruct((n, d), jnp.float32),
        compiler_params=pltpu.CompilerParams(use_tc_tiling_on_sc=False),
        scratch_types=[
            pltpu.VMEM((2, c_rows), jnp.int32),
            pltpu.VMEM((2, c_rows), jnp.int32),
            pltpu.VMEM((2, c_rows, d), jnp.float32),
            pltpu.VMEM((2, d), jnp.float32),
            pltpu.SemaphoreType.DMA,
            pltpu.SemaphoreType.DMA,
            pltpu.SemaphoreType.DMA,
            pltpu.SemaphoreType.DMA,
        ],
    )
    def body(idx_hbm, table_hbm, beg_hbm, out_hbm, idx_raw, idx_safe, rows,
             beg_v, gsem0, gsem1, wsem0, wsem1):
        gsem = (gsem0, gsem1)
        wsem = (wsem0, wsem1)
        wid = lax.axis_index("s") * NC + lax.axis_index("c")
        base = wid * per_w
        pltpu.sync_copy(beg_hbm, beg_v)

        def prep(gl, b):
            gc = jnp.minimum(gl, n_chunks - 1)
            off = base + gc * c_rows
            pltpu.sync_copy(idx_hbm.at[pl.ds(off, c_rows)], idx_raw.at[b])

            def remap(i, macc):
                v = idx_raw[b, pl.ds(i * LANES, LANES)]
                safe = jnp.minimum(v, pad)
                idx_safe[b, pl.ds(i * LANES, LANES)] = safe
                pltpu.async_copy(table_hbm.at[safe],
                                 rows.at[b, pl.ds(i * LANES, LANES)],
                                 gsem[b])
                return jnp.maximum(macc, v)

            macc = lax.fori_loop(0, c_rows // LANES, remap,
                                 jnp.zeros((LANES,), jnp.int32))
            mxs = macc[0]
            for k in range(1, LANES):
                mxs = jnp.maximum(mxs, macc[k])
            return mxs

        def fixup(b, mxs):
            @pl.when(mxs >= start)
            def _fix():
                b0 = [beg_v[0, pl.ds(k * LANES, LANES)]
                      for k in range(d // LANES)]
                b1 = [beg_v[1, pl.ds(k * LANES, LANES)]
                      for k in range(d // LANES)]

                def fv(j, c2):
                    vraw = idx_raw[b, pl.ds(j * LANES, LANES)]
                    for lane in range(LANES):
                        sv = vraw[lane]
                        r = j * LANES + lane

                        @pl.when(sv == start)
                        def _s(r=r):
                            for k in range(d // LANES):
                                rows[b, r, pl.ds(k * LANES, LANES)] = b0[k]

                        @pl.when(sv == start + 1)
                        def _e(r=r):
                            for k in range(d // LANES):
                                rows[b, r, pl.ds(k * LANES, LANES)] = b1[k]
                    return c2

                lax.fori_loop(0, c_rows // LANES, fv, 0)

        mx0 = prep(0, 0)

        @pl.loop(0, n_chunks, step=2, init_carry=mx0)
        def _pipe(g, mx_cur):
            for b in range(2):
                gl = g + b
                nb = 1 - b

                @pl.when(gl > 0)
                def _drain(gl=gl, nb=nb):
                    off_prev = base + (gl - 1) * c_rows
                    pltpu.make_async_copy(
                        rows.at[nb], out_hbm.at[pl.ds(off_prev, c_rows)],
                        wsem[nb]).wait()

                mx_next = prep(gl + 1, nb)
                pltpu.make_async_copy(table_hbm.at[pl.ds(0, c_rows)],
                                      rows.at[b], gsem[b]).wait()
                fixup(b, mx_cur)
                off = base + gl * c_rows
                pltpu.async_copy(rows.at[b],
                                 out_hbm.at[pl.ds(off, c_rows)], wsem[b])
                mx_cur = mx_next
            return mx_cur

        pltpu.make_async_copy(table_hbm.at[pl.ds(0, c_rows)],
                              rows.at[0], gsem[0]).wait()
        off_last = base + (n_chunks - 1) * c_rows
        pltpu.make_async_copy(rows.at[1],
                              out_hbm.at[pl.ds(off_last, c_rows)],
                              wsem[1]).wait()

    return body


def kernel(idxes, table, beg_end):
    b, h = idxes.shape
    v_rows, d = table.shape
    n = b * h
    flat = idxes.reshape(n)
    out = _build(n, v_rows, d)(flat, table, beg_end)
    return out.reshape(b, h, d)
